# initial kernel scaffold (unmeasured)
import jax
import jax.numpy as jnp
from jax import lax
from jax.experimental import pallas as pl
from jax.experimental.pallas import tpu as pltpu

N_DEV = 32
N_GLOBAL = 16384
EPS = 1e-5


def kernel(x, gamma):
    m, n_per = x.shape
    g_row = gamma.reshape(1, n_per)

    def body(x_ref, g_ref, out_ref, part_ref, sends, recvs):
        me = lax.axis_index("i")

        xf = x_ref[:, :].astype(jnp.float32)
        partial = jnp.sum(xf * xf, axis=1)
        part_ref[0, :] = partial

        for d in range(1, N_DEV):
            rdma = pltpu.make_async_remote_copy(
                src_ref=part_ref.at[pl.ds(0, 1)],
                dst_ref=part_ref.at[pl.ds(d, 1)],
                send_sem=sends.at[d],
                recv_sem=recvs.at[d],
                device_id=((me + d) % N_DEV,),
                device_id_type=pl.DeviceIdType.MESH,
            )
            rdma.start()

        for d in range(1, N_DEV):
            rdma = pltpu.make_async_remote_copy(
                src_ref=part_ref.at[pl.ds(0, 1)],
                dst_ref=part_ref.at[pl.ds(d, 1)],
                send_sem=sends.at[d],
                recv_sem=recvs.at[d],
                device_id=((me + d) % N_DEV,),
                device_id_type=pl.DeviceIdType.MESH,
            )
            rdma.wait_recv()
        for d in range(1, N_DEV):
            rdma = pltpu.make_async_remote_copy(
                src_ref=part_ref.at[pl.ds(0, 1)],
                dst_ref=part_ref.at[pl.ds(d, 1)],
                send_sem=sends.at[d],
                recv_sem=recvs.at[d],
                device_id=((me + d) % N_DEV,),
                device_id_type=pl.DeviceIdType.MESH,
            )
            rdma.wait_send()

        total = jnp.sum(part_ref[:, :], axis=0)
        inv = (1.0 / jnp.sqrt(total / N_GLOBAL + EPS)).reshape(m, 1)
        out_ref[:, :] = xf * inv * g_ref[0:1, :].astype(jnp.float32)

    return pl.pallas_call(
        body,
        out_shape=jax.ShapeDtypeStruct((m, n_per), jnp.float32),
        in_specs=[
            pl.BlockSpec(memory_space=pltpu.VMEM),
            pl.BlockSpec(memory_space=pltpu.VMEM),
        ],
        out_specs=pl.BlockSpec(memory_space=pltpu.VMEM),
        scratch_shapes=[
            pltpu.VMEM((N_DEV, m), jnp.float32),
            pltpu.SemaphoreType.DMA((N_DEV,)),
            pltpu.SemaphoreType.DMA((N_DEV,)),
        ],
        compiler_params=pltpu.CompilerParams(collective_id=0),
    )(x, g_row)


# baseline (device time: 22893 ns/iter reference)
import jax
import jax.numpy as jnp
from jax import lax
from jax.experimental import pallas as pl
from jax.experimental.pallas import tpu as pltpu

N_DEV = 32
N_GLOBAL = 16384
EPS = 1e-5


def kernel(x, gamma):
    m, n_per = x.shape
    g_row = gamma.reshape(1, n_per)

    def body(x_ref, g_ref, out_ref, part_ref, sends, recvs):
        me = lax.axis_index("i")

        xf = x_ref[:, :].astype(jnp.float32)
        partial = jnp.sum(xf * xf, axis=1)
        part_ref[0, :] = partial

        for d in range(1, N_DEV):
            rdma = pltpu.make_async_remote_copy(
                src_ref=part_ref.at[pl.ds(0, 1)],
                dst_ref=part_ref.at[pl.ds(d, 1)],
                send_sem=sends.at[d],
                recv_sem=recvs.at[d],
                device_id=((me + d) % N_DEV,),
                device_id_type=pl.DeviceIdType.MESH,
            )
            rdma.start()

        for d in range(1, N_DEV):
            rdma = pltpu.make_async_remote_copy(
                src_ref=part_ref.at[pl.ds(0, 1)],
                dst_ref=part_ref.at[pl.ds(d, 1)],
                send_sem=sends.at[d],
                recv_sem=recvs.at[d],
                device_id=((me + d) % N_DEV,),
                device_id_type=pl.DeviceIdType.MESH,
            )
            rdma.wait_recv()
        for d in range(1, N_DEV):
            rdma = pltpu.make_async_remote_copy(
                src_ref=part_ref.at[pl.ds(0, 1)],
                dst_ref=part_ref.at[pl.ds(d, 1)],
                send_sem=sends.at[d],
                recv_sem=recvs.at[d],
                device_id=((me + d) % N_DEV,),
                device_id_type=pl.DeviceIdType.MESH,
            )
            rdma.wait_send()

        total = jnp.sum(part_ref[:, :], axis=0)
        inv = (1.0 / jnp.sqrt(total / N_GLOBAL + EPS)).reshape(m, 1)
        out_ref[:, :] = xf * inv * g_ref[0:1, :].astype(jnp.float32)

    return pl.pallas_call(
        body,
        out_shape=jax.ShapeDtypeStruct((m, n_per), jnp.float32),
        in_specs=[
            pl.BlockSpec(memory_space=pltpu.VMEM),
            pl.BlockSpec(memory_space=pltpu.VMEM),
        ],
        out_specs=pl.BlockSpec(memory_space=pltpu.VMEM),
        scratch_shapes=[
            pltpu.VMEM((N_DEV, m), jnp.float32),
            pltpu.SemaphoreType.DMA((N_DEV,)),
            pltpu.SemaphoreType.DMA((N_DEV,)),
        ],
    )(x, g_row)


# device time: 14883 ns/iter; 1.5382x vs baseline; 1.5382x over previous
import jax
import jax.numpy as jnp
from jax import lax
from jax.experimental import pallas as pl
from jax.experimental.pallas import tpu as pltpu

N_DEV = 32
N_GLOBAL = 16384
EPS = 1e-5


def kernel(x, gamma):
    m, n_per = x.shape
    g_row = gamma.reshape(1, n_per)

    def body(x_ref, g_ref, out_ref, part_ref, sends, recvs):
        me = lax.axis_index("i")

        def peer_rdma(d):
            return pltpu.make_async_remote_copy(
                src_ref=part_ref.at[pl.ds(0, 1)],
                dst_ref=part_ref.at[pl.ds(d, 1)],
                send_sem=sends.at[d],
                recv_sem=recvs.at[d],
                device_id=((me + d) % N_DEV,),
                device_id_type=pl.DeviceIdType.MESH,
            )

        barrier = pltpu.get_barrier_semaphore()
        for d in range(1, N_DEV):
            pl.semaphore_signal(
                barrier,
                inc=1,
                device_id=((me + d) % N_DEV,),
                device_id_type=pl.DeviceIdType.MESH,
            )

        xf = x_ref[:, :].astype(jnp.float32)
        partial = jnp.sum(xf * xf, axis=1)
        part_ref[0, :] = partial

        pl.semaphore_wait(barrier, N_DEV - 1)
        for d in range(1, N_DEV):
            peer_rdma(d).start()

        y = xf * g_ref[0:1, :].astype(jnp.float32)

        for d in range(1, N_DEV):
            peer_rdma(d).wait_recv()

        total = jnp.sum(part_ref[:, :], axis=0)
        inv = (1.0 / jnp.sqrt(total / N_GLOBAL + EPS)).reshape(m, 1)
        out_ref[:, :] = y * inv

        for d in range(1, N_DEV):
            peer_rdma(d).wait_send()

    return pl.pallas_call(
        body,
        out_shape=jax.ShapeDtypeStruct((m, n_per), jnp.float32),
        in_specs=[
            pl.BlockSpec(memory_space=pltpu.VMEM),
            pl.BlockSpec(memory_space=pltpu.VMEM),
        ],
        out_specs=pl.BlockSpec(memory_space=pltpu.VMEM),
        scratch_shapes=[
            pltpu.VMEM((N_DEV, m), jnp.float32),
            pltpu.SemaphoreType.DMA((N_DEV,)),
            pltpu.SemaphoreType.DMA((N_DEV,)),
        ],
        compiler_params=pltpu.CompilerParams(collective_id=0),
    )(x, g_row)
